# baseline (speedup 1.0000x reference)
import jax
import jax.numpy as jnp
from jax import lax
from jax.experimental import pallas as pl
from jax.experimental.pallas import tpu as pltpu

B = 16
L = 4096
TOKEN_DIM = 64
STRIDE = 56
NT = 73
OUT_W = NT * TOKEN_DIM
FULL_TILES = OUT_W // 128


def _tokenize_tc_body(in_ref, out_ref, flat_ref):
    lane = lax.broadcasted_iota(jnp.int32, (B, 128), 1)
    first_half = lane < TOKEN_DIM
    for k in range(FULL_TILES):
        a = in_ref[:, 112 * k:112 * k + 128]
        if k == 0:
            b = jnp.roll(a, 8, axis=1)
        else:
            b = in_ref[:, 112 * k - 8:112 * k + 120]
        flat_ref[:, 128 * k:128 * k + 128] = jnp.where(first_half, a, b)
    flat_ref[:, FULL_TILES * 128:] = in_ref[:, STRIDE * (NT - 1):]
    for w in range(NT // 8):
        out_ref[:, 8 * w:8 * w + 8, :] = flat_ref[
            :, 512 * w:512 * w + 512
        ].reshape(B, 8, TOKEN_DIM)
    out_ref[:, NT - 1:NT, :] = flat_ref[:, OUT_W - TOKEN_DIM:].reshape(
        B, 1, TOKEN_DIM
    )


def kernel(inputs):
    return pl.pallas_call(
        _tokenize_tc_body,
        out_shape=jax.ShapeDtypeStruct((B, NT, TOKEN_DIM), jnp.float32),
        scratch_shapes=[pltpu.VMEM((B, OUT_W), jnp.float32)],
    )(inputs)


# speedup vs baseline: 1.0010x; 1.0010x over previous
import jax
import jax.numpy as jnp
from jax import lax
from jax.experimental import pallas as pl
from jax.experimental.pallas import tpu as pltpu

B = 16
L = 4096
TOKEN_DIM = 64
STRIDE = 56
NT = 73
OUT_W = NT * TOKEN_DIM
FULL_TILES = OUT_W // 128


def _tokenize_tc_body(in_ref, out_hbm, flat_ref, tiled_ref, sem):
    lane = lax.broadcasted_iota(jnp.int32, (B, 128), 1)
    first_half = lane < TOKEN_DIM
    for k in range(FULL_TILES):
        a = in_ref[:, 112 * k:112 * k + 128]
        if k == 0:
            b = jnp.roll(a, 8, axis=1)
        else:
            b = in_ref[:, 112 * k - 8:112 * k + 120]
        flat_ref[:, 128 * k:128 * k + 128] = jnp.where(first_half, a, b)
    flat_ref[:, FULL_TILES * 128:] = in_ref[:, STRIDE * (NT - 1):]
    for w in range(NT // 8):
        tiled_ref[:, 8 * w:8 * w + 8, :] = flat_ref[
            :, 512 * w:512 * w + 512
        ].reshape(B, 8, TOKEN_DIM)
    tiled_ref[:, NT - 1:NT, :] = flat_ref[:, OUT_W - TOKEN_DIM:].reshape(
        B, 1, TOKEN_DIM
    )
    pltpu.make_async_copy(tiled_ref, out_hbm, sem).start()
    pltpu.make_async_copy(tiled_ref, out_hbm, sem).wait()


def kernel(inputs):
    return pl.pallas_call(
        _tokenize_tc_body,
        out_shape=jax.ShapeDtypeStruct((B, NT, TOKEN_DIM), jnp.float32),
        out_specs=pl.BlockSpec(memory_space=pl.ANY),
        scratch_shapes=[
            pltpu.VMEM((B, OUT_W), jnp.float32),
            pltpu.VMEM((B, NT, TOKEN_DIM), jnp.float32),
            pltpu.SemaphoreType.DMA,
        ],
    )(inputs)


# speedup vs baseline: 1.4880x; 1.4866x over previous
import jax
import jax.numpy as jnp
from jax import lax
from jax.experimental import pallas as pl
from jax.experimental.pallas import tpu as pltpu

B = 16
L = 4096
TOKEN_DIM = 64
STRIDE = 56
NT = 73
OUT_W = NT * TOKEN_DIM
FULL_TILES = OUT_W // 128


def _tokenize_tc_body(in_ref, out_ref, flat_ref, tiled_ref):
    lane = lax.broadcasted_iota(jnp.int32, (B, 128), 1)
    first_half = lane < TOKEN_DIM
    for k in range(FULL_TILES):
        a = in_ref[:, 112 * k:112 * k + 128]
        if k == 0:
            b = jnp.roll(a, 8, axis=1)
        else:
            b = in_ref[:, 112 * k - 8:112 * k + 120]
        flat_ref[:, 128 * k:128 * k + 128] = jnp.where(first_half, a, b)
    flat_ref[:, FULL_TILES * 128:] = in_ref[:, STRIDE * (NT - 1):]
    for w in range(NT // 8):
        tiled_ref[:, 8 * w:8 * w + 8, :] = flat_ref[
            :, 512 * w:512 * w + 512
        ].reshape(B, 8, TOKEN_DIM)
    tiled_ref[:, NT - 1:NT, :] = flat_ref[:, OUT_W - TOKEN_DIM:].reshape(
        B, 1, TOKEN_DIM
    )
    row = lax.broadcasted_iota(jnp.int32, (NT, NT), 0)
    col = lax.broadcasted_iota(jnp.int32, (NT, NT), 1)
    eye = (row == col).astype(jnp.float32)
    for b in range(B):
        y = tiled_ref[b]
        out_ref[b] = lax.dot_general(
            y, eye, (((0,), (0,)), ((), ())),
            precision=lax.Precision.HIGHEST,
        )


def kernel(inputs):
    t_out = pl.pallas_call(
        _tokenize_tc_body,
        out_shape=jax.ShapeDtypeStruct((B, TOKEN_DIM, NT), jnp.float32),
        scratch_shapes=[
            pltpu.VMEM((B, OUT_W), jnp.float32),
            pltpu.VMEM((B, NT, TOKEN_DIM), jnp.float32),
        ],
    )(inputs)
    return t_out.transpose(0, 2, 1)


# speedup vs baseline: 1.8037x; 1.2121x over previous
import jax
import jax.numpy as jnp
from jax import lax
from jax.experimental import pallas as pl
from jax.experimental.pallas import tpu as pltpu

B = 16
L = 4096
TOKEN_DIM = 64
STRIDE = 56
NT = 73
OUT_W = NT * TOKEN_DIM
FULL_TILES = OUT_W // 128


def _tokenize_tc_body(in_ref, out_ref, flat_ref, tiled_ref):
    lane = lax.broadcasted_iota(jnp.int32, (B, 128), 1)
    first_half = lane < TOKEN_DIM
    for k in range(FULL_TILES):
        a = in_ref[:, 112 * k:112 * k + 128]
        if k == 0:
            b = jnp.roll(a, 8, axis=1)
        else:
            b = in_ref[:, 112 * k - 8:112 * k + 120]
        flat_ref[:, 128 * k:128 * k + 128] = jnp.where(first_half, a, b)
    flat_ref[:, FULL_TILES * 128:] = in_ref[:, STRIDE * (NT - 1):]
    for w in range(NT // 8):
        tiled_ref[:, 8 * w:8 * w + 8, :] = flat_ref[
            :, 512 * w:512 * w + 512
        ].reshape(B, 8, TOKEN_DIM)
    tiled_ref[:, NT - 1:NT, :] = flat_ref[:, OUT_W - TOKEN_DIM:].reshape(
        B, 1, TOKEN_DIM
    )
    for b in range(B):
        out_ref[b] = jnp.transpose(tiled_ref[b])


def kernel(inputs):
    t_out = pl.pallas_call(
        _tokenize_tc_body,
        out_shape=jax.ShapeDtypeStruct((B, TOKEN_DIM, NT), jnp.float32),
        scratch_shapes=[
            pltpu.VMEM((B, OUT_W), jnp.float32),
            pltpu.VMEM((B, NT, TOKEN_DIM), jnp.float32),
        ],
    )(inputs)
    return t_out.transpose(0, 2, 1)
